# Initial kernel scaffold; baseline (speedup 1.0000x reference)
import jax
import jax.numpy as jnp
from jax.experimental import pallas as pl


def kernel(x, edge_index, W1, b1, W2, b2, W_out, b_out):
    raise NotImplementedError("write your pallas kernel here")


# speedup vs baseline: 12.3010x; 12.3010x over previous
import functools

import jax
import jax.numpy as jnp
from jax import lax
from jax.experimental import pallas as pl
from jax.experimental.pallas import tpu as pltpu
from jax.experimental.pallas import tpu_sc as plsc

N = 10000
E = 320000
D = 128

NC = 2
NS = 16
NW = NC * NS

E_PER_TILE = E // NW
CHUNK = 80
STEPS = E_PER_TILE // CHUNK

N_ROWS = 10240
ROWS_PER_TILE = N_ROWS // NS
ZCHUNK = 128

_MESH = plsc.VectorSubcoreMesh(core_axis_name="c", subcore_axis_name="s")


def _zero_vmem_2d(ref, rows, cols):
  zero = jnp.zeros((16,), jnp.float32)

  def body(r, carry):
    for j in range(cols // 16):
      ref[r, pl.ds(j * 16, 16)] = zero
    return carry

  lax.fori_loop(0, rows, body, 0)


def _make_deg():
  @functools.partial(
      pl.kernel,
      out_type=jax.ShapeDtypeStruct((NC, N_ROWS, D), jnp.float32),
      mesh=_MESH,
      scratch_types=[
          pltpu.VMEM((CHUNK,), jnp.int32),
          pltpu.VMEM((CHUNK, D), jnp.float32),
          pltpu.VMEM((ZCHUNK, D), jnp.float32),
          pltpu.VMEM_SHARED((N_ROWS, D), jnp.float32),
      ],
  )
  def deg_kernel(ones_hbm, dst_hbm, out_hbm, idx_v, ones_v, zrow_v, acc_sh):
    c = lax.axis_index("c")
    s = lax.axis_index("s")
    wid = s * NC + c

    pltpu.sync_copy(ones_hbm, ones_v)
    _zero_vmem_2d(zrow_v, ZCHUNK, D)
    for k in range(ROWS_PER_TILE // ZCHUNK):
      pltpu.sync_copy(
          zrow_v, acc_sh.at[pl.ds(s * ROWS_PER_TILE + k * ZCHUNK, ZCHUNK)]
      )
    plsc.subcore_barrier()

    base = wid * E_PER_TILE

    def step(i, carry):
      pltpu.sync_copy(dst_hbm.at[pl.ds(base + i * CHUNK, CHUNK)], idx_v)
      pltpu.sync_copy(ones_v, acc_sh.at[idx_v], add=True)
      return carry

    lax.fori_loop(0, STEPS, step, 0)
    plsc.subcore_barrier()

    pltpu.sync_copy(
        acc_sh.at[pl.ds(s * ROWS_PER_TILE, ROWS_PER_TILE)],
        out_hbm.at[c, pl.ds(s * ROWS_PER_TILE, ROWS_PER_TILE)],
    )

  return deg_kernel


def _make_agg():
  @functools.partial(
      pl.kernel,
      out_type=jax.ShapeDtypeStruct((NC, N_ROWS, D), jnp.float32),
      mesh=_MESH,
      scratch_types=[
          pltpu.VMEM((CHUNK,), jnp.int32),
          pltpu.VMEM((CHUNK,), jnp.int32),
          pltpu.VMEM((CHUNK, D), jnp.float32),
          pltpu.VMEM((ZCHUNK, D), jnp.float32),
          pltpu.VMEM_SHARED((N_ROWS, D), jnp.float32),
          pltpu.SemaphoreType.DMA,
      ],
  )
  def agg_kernel(y_hbm, src_hbm, dst_hbm, out_hbm, src_v, dst_v, rows_v,
                 zrow_v, acc_sh, sem):
    c = lax.axis_index("c")
    s = lax.axis_index("s")
    wid = s * NC + c

    _zero_vmem_2d(zrow_v, ZCHUNK, D)
    for k in range(ROWS_PER_TILE // ZCHUNK):
      pltpu.sync_copy(
          zrow_v, acc_sh.at[pl.ds(s * ROWS_PER_TILE + k * ZCHUNK, ZCHUNK)]
      )
    plsc.subcore_barrier()

    base = wid * E_PER_TILE

    def step(i, carry):
      off = base + i * CHUNK
      pltpu.sync_copy(src_hbm.at[pl.ds(off, CHUNK)], src_v)
      pltpu.sync_copy(dst_hbm.at[pl.ds(off, CHUNK)], dst_v)
      pltpu.async_copy(y_hbm.at[src_v], rows_v, sem).wait()
      pltpu.sync_copy(rows_v, acc_sh.at[dst_v], add=True)
      return carry

    lax.fori_loop(0, STEPS, step, 0)
    plsc.subcore_barrier()

    pltpu.sync_copy(
        acc_sh.at[pl.ds(s * ROWS_PER_TILE, ROWS_PER_TILE)],
        out_hbm.at[c, pl.ds(s * ROWS_PER_TILE, ROWS_PER_TILE)],
    )

  return agg_kernel


_deg_call = _make_deg()
_agg_call = _make_agg()


R_BLK = 2000
GRID = N // R_BLK


def _tc1_body(x_ref, w_ref, dis_ref, xw_ref, y_ref):
  xw = jnp.dot(x_ref[...], w_ref[...], preferred_element_type=jnp.float32)
  xw_ref[...] = xw
  y_ref[...] = xw * dis_ref[...]


def _tc1(x, w1, dis2d):
  return pl.pallas_call(
      _tc1_body,
      grid=(GRID,),
      in_specs=[
          pl.BlockSpec((R_BLK, D), lambda i: (i, 0)),
          pl.BlockSpec((D, D), lambda i: (0, 0)),
          pl.BlockSpec((R_BLK, 1), lambda i: (i, 0)),
      ],
      out_specs=[
          pl.BlockSpec((R_BLK, D), lambda i: (i, 0)),
          pl.BlockSpec((R_BLK, D), lambda i: (i, 0)),
      ],
      out_shape=[
          jax.ShapeDtypeStruct((N, D), jnp.float32),
          jax.ShapeDtypeStruct((N, D), jnp.float32),
      ],
  )(x, w1, dis2d)


def _tc2_body(p_ref, xw_ref, dis_ref, b_ref, w_ref, xw2_ref, y2_ref):
  dis = dis_ref[...]
  agg = p_ref[0] + p_ref[1]
  h = jnp.maximum(dis * agg + (dis * dis) * xw_ref[...] + b_ref[...], 0.0)
  xw2 = jnp.dot(h, w_ref[...], preferred_element_type=jnp.float32)
  xw2_ref[...] = xw2
  y2_ref[...] = xw2 * dis


def _tc2(p, xw1, dis2d, b1, w2):
  return pl.pallas_call(
      _tc2_body,
      grid=(GRID,),
      in_specs=[
          pl.BlockSpec((NC, R_BLK, D), lambda i: (0, i, 0)),
          pl.BlockSpec((R_BLK, D), lambda i: (i, 0)),
          pl.BlockSpec((R_BLK, 1), lambda i: (i, 0)),
          pl.BlockSpec((1, D), lambda i: (0, 0)),
          pl.BlockSpec((D, D), lambda i: (0, 0)),
      ],
      out_specs=[
          pl.BlockSpec((R_BLK, D), lambda i: (i, 0)),
          pl.BlockSpec((R_BLK, D), lambda i: (i, 0)),
      ],
      out_shape=[
          jax.ShapeDtypeStruct((N, D), jnp.float32),
          jax.ShapeDtypeStruct((N, D), jnp.float32),
      ],
  )(p, xw1, dis2d, b1, w2)


def _tc3_body(p_ref, xw_ref, dis_ref, b_ref, wout_ref, bout_ref, out_ref):
  dis = dis_ref[...]
  agg = p_ref[0] + p_ref[1]
  h = jnp.maximum(dis * agg + (dis * dis) * xw_ref[...] + b_ref[...], 0.0)
  logit = jnp.dot(h, wout_ref[...], preferred_element_type=jnp.float32)
  logit = logit + bout_ref[...]
  out_ref[...] = 1.0 / (1.0 + jnp.exp(-logit))


def _tc3(p, xw2, dis2d, b2, w_out, b_out):
  return pl.pallas_call(
      _tc3_body,
      grid=(GRID,),
      in_specs=[
          pl.BlockSpec((NC, R_BLK, D), lambda i: (0, i, 0)),
          pl.BlockSpec((R_BLK, D), lambda i: (i, 0)),
          pl.BlockSpec((R_BLK, 1), lambda i: (i, 0)),
          pl.BlockSpec((1, D), lambda i: (0, 0)),
          pl.BlockSpec((D, 1), lambda i: (0, 0)),
          pl.BlockSpec((1, 1), lambda i: (0, 0)),
      ],
      out_specs=pl.BlockSpec((R_BLK, 1), lambda i: (i, 0)),
      out_shape=jax.ShapeDtypeStruct((N, 1), jnp.float32),
  )(p, xw2, dis2d, b2, w_out, b_out)


def kernel(x, edge_index, W1, b1, W2, b2, W_out, b_out):
  src = edge_index[0].astype(jnp.int32)
  dst = edge_index[1].astype(jnp.int32)

  ones_rows = jnp.ones((CHUNK, D), jnp.float32)
  deg_p = _deg_call(ones_rows, dst)
  deg = 1.0 + deg_p[0, :N, 0] + deg_p[1, :N, 0]
  dis2d = lax.rsqrt(deg)[:, None]

  xw1, y1 = _tc1(x, W1, dis2d)
  p1 = _agg_call(y1, src, dst)
  xw2, y2 = _tc2(p1, xw1, dis2d, b1.reshape(1, D), W2)
  p2 = _agg_call(y2, src, dst)
  out = _tc3(p2, xw2, dis2d, b2.reshape(1, D), W_out, b_out.reshape(1, 1))
  return out[:, 0]


# speedup vs baseline: 20.9642x; 1.7043x over previous
import functools

import jax
import jax.numpy as jnp
from jax import lax
from jax.experimental import pallas as pl
from jax.experimental.pallas import tpu as pltpu
from jax.experimental.pallas import tpu_sc as plsc

N = 10000
E = 320000
D = 128

NC = 2
NS = 16
NW = NC * NS

E_PER_TILE = E // NW
CHUNK = 40
STEPS = E_PER_TILE // CHUNK

N_ROWS = 10240
ROWS_PER_TILE = N_ROWS // NS
ZCHUNK = 32

_MESH = plsc.VectorSubcoreMesh(core_axis_name="c", subcore_axis_name="s")


def _zero_vmem_2d(ref, rows, cols):
  zero = jnp.zeros((16,), jnp.float32)

  def body(r, carry):
    for j in range(cols // 16):
      ref[r, pl.ds(j * 16, 16)] = zero
    return carry

  lax.fori_loop(0, rows, body, 0)


def _make_deg():
  @functools.partial(
      pl.kernel,
      out_type=jax.ShapeDtypeStruct((NC, N_ROWS, D), jnp.float32),
      mesh=_MESH,
      scratch_types=[
          pltpu.VMEM((CHUNK,), jnp.int32),
          pltpu.VMEM((CHUNK, D), jnp.float32),
          pltpu.VMEM((ZCHUNK, D), jnp.float32),
          pltpu.VMEM_SHARED((N_ROWS, D), jnp.float32),
      ],
  )
  def deg_kernel(ones_hbm, dst_hbm, out_hbm, idx_v, ones_v, zrow_v, acc_sh):
    c = lax.axis_index("c")
    s = lax.axis_index("s")
    wid = s * NC + c

    pltpu.sync_copy(ones_hbm, ones_v)
    _zero_vmem_2d(zrow_v, ZCHUNK, D)
    for k in range(ROWS_PER_TILE // ZCHUNK):
      pltpu.sync_copy(
          zrow_v, acc_sh.at[pl.ds(s * ROWS_PER_TILE + k * ZCHUNK, ZCHUNK)]
      )
    plsc.subcore_barrier()

    base = wid * E_PER_TILE

    def step(i, carry):
      pltpu.sync_copy(dst_hbm.at[pl.ds(base + i * CHUNK, CHUNK)], idx_v)
      pltpu.sync_copy(ones_v, acc_sh.at[idx_v], add=True)
      return carry

    lax.fori_loop(0, STEPS, step, 0)
    plsc.subcore_barrier()

    pltpu.sync_copy(
        acc_sh.at[pl.ds(s * ROWS_PER_TILE, ROWS_PER_TILE)],
        out_hbm.at[c, pl.ds(s * ROWS_PER_TILE, ROWS_PER_TILE)],
    )

  return deg_kernel


NBUF = 5


def _make_agg():
  scratch = (
      [pltpu.VMEM((CHUNK,), jnp.int32) for _ in range(NBUF)]
      + [pltpu.VMEM((CHUNK,), jnp.int32) for _ in range(NBUF)]
      + [pltpu.VMEM((CHUNK, D), jnp.float32) for _ in range(NBUF)]
      + [pltpu.VMEM((ZCHUNK, D), jnp.float32)]
      + [pltpu.VMEM_SHARED((N_ROWS, D), jnp.float32)]
      + [pltpu.SemaphoreType.DMA for _ in range(3 * NBUF)]
  )

  @functools.partial(
      pl.kernel,
      out_type=jax.ShapeDtypeStruct((NC, N_ROWS, D), jnp.float32),
      mesh=_MESH,
      scratch_types=scratch,
  )
  def agg_kernel(y_hbm, src_hbm, dst_hbm, out_hbm, *refs):
    srcv = refs[0:NBUF]
    dstv = refs[NBUF:2 * NBUF]
    rows = refs[2 * NBUF:3 * NBUF]
    zrow_v = refs[3 * NBUF]
    acc_sh = refs[3 * NBUF + 1]
    isem = refs[3 * NBUF + 2:3 * NBUF + 2 + NBUF]
    gsem = refs[3 * NBUF + 2 + NBUF:3 * NBUF + 2 + 2 * NBUF]
    ssem = refs[3 * NBUF + 2 + 2 * NBUF:3 * NBUF + 2 + 3 * NBUF]

    c = lax.axis_index("c")
    s = lax.axis_index("s")
    wid = s * NC + c

    _zero_vmem_2d(zrow_v, ZCHUNK, D)
    for k in range(ROWS_PER_TILE // ZCHUNK):
      pltpu.sync_copy(
          zrow_v, acc_sh.at[pl.ds(s * ROWS_PER_TILE + k * ZCHUNK, ZCHUNK)]
      )
    plsc.subcore_barrier()

    base = wid * E_PER_TILE

    def i_start(i, b):
      off = base + i * CHUNK
      pltpu.async_copy(src_hbm.at[pl.ds(off, CHUNK)], srcv[b], isem[b])
      pltpu.async_copy(dst_hbm.at[pl.ds(off, CHUNK)], dstv[b], isem[b])

    def iw_gs(i, b):
      off = base + i * CHUNK
      pltpu.make_async_copy(src_hbm.at[pl.ds(off, CHUNK)], srcv[b], isem[b]).wait()
      pltpu.make_async_copy(dst_hbm.at[pl.ds(off, CHUNK)], dstv[b], isem[b]).wait()
      pltpu.async_copy(y_hbm.at[srcv[b]], rows[b], gsem[b])

    def gw_ss(i, b):
      pltpu.make_async_copy(y_hbm.at[srcv[b]], rows[b], gsem[b]).wait()
      pltpu.async_copy(rows[b], acc_sh.at[dstv[b]], ssem[b], add=True)

    def s_wait(i, b):
      pltpu.make_async_copy(rows[b], acc_sh.at[dstv[b]], ssem[b]).wait()

    def full_step(t, b):
      s_wait(t - NBUF, b)
      i_start(t, b)
      iw_gs(t - 1, (b + NBUF - 1) % NBUF)
      gw_ss(t - 3, (b + NBUF - 3) % NBUF)

    for t in range(NBUF):
      i_start(t, t % NBUF)
      if t - 1 >= 0:
        iw_gs(t - 1, (t - 1) % NBUF)
      if t - 3 >= 0:
        gw_ss(t - 3, (t - 3) % NBUF)

    def batch(j, carry):
      for b in range(NBUF):
        full_step(j * NBUF + b, b)
      return carry

    lax.fori_loop(1, STEPS // NBUF, batch, 0)

    for t in range(STEPS, STEPS + NBUF):
      s_wait(t - NBUF, (t - NBUF) % NBUF)
      if t - 1 < STEPS:
        iw_gs(t - 1, (t - 1) % NBUF)
      if t - 3 < STEPS:
        gw_ss(t - 3, (t - 3) % NBUF)

    plsc.subcore_barrier()

    pltpu.sync_copy(
        acc_sh.at[pl.ds(s * ROWS_PER_TILE, ROWS_PER_TILE)],
        out_hbm.at[c, pl.ds(s * ROWS_PER_TILE, ROWS_PER_TILE)],
    )

  return agg_kernel


_deg_call = _make_deg()
_agg_call = _make_agg()


R_BLK = 2000
GRID = N // R_BLK


def _tc1_body(x_ref, w_ref, dis_ref, xw_ref, y_ref):
  xw = jnp.dot(x_ref[...], w_ref[...], preferred_element_type=jnp.float32)
  xw_ref[...] = xw
  y_ref[...] = xw * dis_ref[...]


def _tc1(x, w1, dis2d):
  return pl.pallas_call(
      _tc1_body,
      grid=(GRID,),
      in_specs=[
          pl.BlockSpec((R_BLK, D), lambda i: (i, 0)),
          pl.BlockSpec((D, D), lambda i: (0, 0)),
          pl.BlockSpec((R_BLK, 1), lambda i: (i, 0)),
      ],
      out_specs=[
          pl.BlockSpec((R_BLK, D), lambda i: (i, 0)),
          pl.BlockSpec((R_BLK, D), lambda i: (i, 0)),
      ],
      out_shape=[
          jax.ShapeDtypeStruct((N, D), jnp.float32),
          jax.ShapeDtypeStruct((N, D), jnp.float32),
      ],
  )(x, w1, dis2d)


def _tc2_body(p_ref, xw_ref, dis_ref, b_ref, w_ref, xw2_ref, y2_ref):
  dis = dis_ref[...]
  agg = p_ref[0] + p_ref[1]
  h = jnp.maximum(dis * agg + (dis * dis) * xw_ref[...] + b_ref[...], 0.0)
  xw2 = jnp.dot(h, w_ref[...], preferred_element_type=jnp.float32)
  xw2_ref[...] = xw2
  y2_ref[...] = xw2 * dis


def _tc2(p, xw1, dis2d, b1, w2):
  return pl.pallas_call(
      _tc2_body,
      grid=(GRID,),
      in_specs=[
          pl.BlockSpec((NC, R_BLK, D), lambda i: (0, i, 0)),
          pl.BlockSpec((R_BLK, D), lambda i: (i, 0)),
          pl.BlockSpec((R_BLK, 1), lambda i: (i, 0)),
          pl.BlockSpec((1, D), lambda i: (0, 0)),
          pl.BlockSpec((D, D), lambda i: (0, 0)),
      ],
      out_specs=[
          pl.BlockSpec((R_BLK, D), lambda i: (i, 0)),
          pl.BlockSpec((R_BLK, D), lambda i: (i, 0)),
      ],
      out_shape=[
          jax.ShapeDtypeStruct((N, D), jnp.float32),
          jax.ShapeDtypeStruct((N, D), jnp.float32),
      ],
  )(p, xw1, dis2d, b1, w2)


def _tc3_body(p_ref, xw_ref, dis_ref, b_ref, wout_ref, bout_ref, out_ref):
  dis = dis_ref[...]
  agg = p_ref[0] + p_ref[1]
  h = jnp.maximum(dis * agg + (dis * dis) * xw_ref[...] + b_ref[...], 0.0)
  logit = jnp.dot(h, wout_ref[...], preferred_element_type=jnp.float32)
  logit = logit + bout_ref[...]
  out_ref[...] = 1.0 / (1.0 + jnp.exp(-logit))


def _tc3(p, xw2, dis2d, b2, w_out, b_out):
  return pl.pallas_call(
      _tc3_body,
      grid=(GRID,),
      in_specs=[
          pl.BlockSpec((NC, R_BLK, D), lambda i: (0, i, 0)),
          pl.BlockSpec((R_BLK, D), lambda i: (i, 0)),
          pl.BlockSpec((R_BLK, 1), lambda i: (i, 0)),
          pl.BlockSpec((1, D), lambda i: (0, 0)),
          pl.BlockSpec((D, 1), lambda i: (0, 0)),
          pl.BlockSpec((1, 1), lambda i: (0, 0)),
      ],
      out_specs=pl.BlockSpec((R_BLK, 1), lambda i: (i, 0)),
      out_shape=jax.ShapeDtypeStruct((N, 1), jnp.float32),
  )(p, xw2, dis2d, b2, w_out, b_out)


def kernel(x, edge_index, W1, b1, W2, b2, W_out, b_out):
  src = edge_index[0].astype(jnp.int32)
  dst = edge_index[1].astype(jnp.int32)

  ones_rows = jnp.ones((CHUNK, D), jnp.float32)
  deg_p = _deg_call(ones_rows, dst)
  deg = 1.0 + deg_p[0, :N, 0] + deg_p[1, :N, 0]
  dis2d = lax.rsqrt(deg)[:, None]

  xw1, y1 = _tc1(x, W1, dis2d)
  p1 = _agg_call(y1, src, dst)
  xw2, y2 = _tc2(p1, xw1, dis2d, b1.reshape(1, D), W2)
  p2 = _agg_call(y2, src, dst)
  out = _tc3(p2, xw2, dis2d, b2.reshape(1, D), W_out, b_out.reshape(1, 1))
  return out[:, 0]


# speedup vs baseline: 27.5716x; 1.3152x over previous
import functools

import jax
import jax.numpy as jnp
from jax import lax
from jax.experimental import pallas as pl
from jax.experimental.pallas import tpu as pltpu
from jax.experimental.pallas import tpu_sc as plsc

N = 10000
E = 320000
D = 128

NC = 2
NS = 16
NW = NC * NS

E_PER_TILE = E // NW
CHUNK = 40
STEPS = E_PER_TILE // CHUNK

N_ROWS = 10240
ROWS_PER_TILE = N_ROWS // NS
ZCHUNK = 32

_MESH = plsc.VectorSubcoreMesh(core_axis_name="c", subcore_axis_name="s")


def _zero_vmem_2d(ref, rows, cols):
  zero = jnp.zeros((16,), jnp.float32)

  def body(r, carry):
    for j in range(cols // 16):
      ref[r, pl.ds(j * 16, 16)] = zero
    return carry

  lax.fori_loop(0, rows, body, 0)


DEG_CHUNK = 80
DEG_STEPS = E_PER_TILE // DEG_CHUNK
DEG_NBUF = 5


def _make_deg():
  scratch = (
      [pltpu.VMEM((DEG_CHUNK,), jnp.int32) for _ in range(DEG_NBUF)]
      + [
          pltpu.VMEM((DEG_CHUNK, D), jnp.float32),
          pltpu.VMEM((ZCHUNK, D), jnp.float32),
          pltpu.VMEM_SHARED((N_ROWS, D), jnp.float32),
      ]
      + [pltpu.SemaphoreType.DMA for _ in range(2 * DEG_NBUF)]
  )

  @functools.partial(
      pl.kernel,
      out_type=jax.ShapeDtypeStruct((NC, N_ROWS, D), jnp.float32),
      mesh=_MESH,
      scratch_types=scratch,
  )
  def deg_kernel(dst_hbm, out_hbm, *refs):
    dstv = refs[0:DEG_NBUF]
    ones_v = refs[DEG_NBUF]
    zrow_v = refs[DEG_NBUF + 1]
    acc_sh = refs[DEG_NBUF + 2]
    isem = refs[DEG_NBUF + 3:DEG_NBUF + 3 + DEG_NBUF]
    ssem = refs[DEG_NBUF + 3 + DEG_NBUF:DEG_NBUF + 3 + 2 * DEG_NBUF]

    c = lax.axis_index("c")
    s = lax.axis_index("s")
    wid = s * NC + c

    one = jnp.full((16,), 1.0, jnp.float32)

    def fill(r, carry):
      for j in range(D // 16):
        ones_v[r, pl.ds(j * 16, 16)] = one
      return carry

    lax.fori_loop(0, DEG_CHUNK, fill, 0)
    _zero_vmem_2d(zrow_v, ZCHUNK, D)
    for k in range(ROWS_PER_TILE // ZCHUNK):
      pltpu.sync_copy(
          zrow_v, acc_sh.at[pl.ds(s * ROWS_PER_TILE + k * ZCHUNK, ZCHUNK)]
      )
    plsc.subcore_barrier()

    base = wid * E_PER_TILE

    def i_start(i, b):
      pltpu.async_copy(
          dst_hbm.at[pl.ds(base + i * DEG_CHUNK, DEG_CHUNK)], dstv[b], isem[b]
      )

    def iw_ss(i, b):
      pltpu.make_async_copy(
          dst_hbm.at[pl.ds(base + i * DEG_CHUNK, DEG_CHUNK)], dstv[b], isem[b]
      ).wait()
      pltpu.async_copy(ones_v, acc_sh.at[dstv[b]], ssem[b], add=True)

    def s_wait(b):
      pltpu.make_async_copy(ones_v, acc_sh.at[dstv[b]], ssem[b]).wait()

    for t in range(DEG_NBUF):
      i_start(t, t)
      if t >= 1:
        iw_ss(t - 1, t - 1)

    def batch(j, carry):
      for b in range(DEG_NBUF):
        t = j * DEG_NBUF + b
        s_wait(b)
        i_start(t, b)
        iw_ss(t - 1, (b + DEG_NBUF - 1) % DEG_NBUF)
      return carry

    lax.fori_loop(1, DEG_STEPS // DEG_NBUF, batch, 0)

    for t in range(DEG_STEPS, DEG_STEPS + DEG_NBUF):
      s_wait((t - DEG_NBUF) % DEG_NBUF)
      if t - 1 < DEG_STEPS:
        iw_ss(t - 1, (t - 1) % DEG_NBUF)

    plsc.subcore_barrier()

    pltpu.sync_copy(
        acc_sh.at[pl.ds(s * ROWS_PER_TILE, ROWS_PER_TILE)],
        out_hbm.at[c, pl.ds(s * ROWS_PER_TILE, ROWS_PER_TILE)],
    )

  return deg_kernel


NBUF = 5


def _make_agg():
  scratch = (
      [pltpu.VMEM((CHUNK,), jnp.int32) for _ in range(NBUF)]
      + [pltpu.VMEM((CHUNK,), jnp.int32) for _ in range(NBUF)]
      + [pltpu.VMEM((CHUNK, D), jnp.float32) for _ in range(NBUF)]
      + [pltpu.VMEM((ZCHUNK, D), jnp.float32)]
      + [pltpu.VMEM_SHARED((N_ROWS, D), jnp.float32)]
      + [pltpu.SemaphoreType.DMA for _ in range(3 * NBUF)]
  )

  @functools.partial(
      pl.kernel,
      out_type=jax.ShapeDtypeStruct((NC, N_ROWS, D), jnp.float32),
      mesh=_MESH,
      scratch_types=scratch,
  )
  def agg_kernel(y_hbm, src_hbm, dst_hbm, out_hbm, *refs):
    srcv = refs[0:NBUF]
    dstv = refs[NBUF:2 * NBUF]
    rows = refs[2 * NBUF:3 * NBUF]
    zrow_v = refs[3 * NBUF]
    acc_sh = refs[3 * NBUF + 1]
    isem = refs[3 * NBUF + 2:3 * NBUF + 2 + NBUF]
    gsem = refs[3 * NBUF + 2 + NBUF:3 * NBUF + 2 + 2 * NBUF]
    ssem = refs[3 * NBUF + 2 + 2 * NBUF:3 * NBUF + 2 + 3 * NBUF]

    c = lax.axis_index("c")
    s = lax.axis_index("s")
    wid = s * NC + c

    _zero_vmem_2d(zrow_v, ZCHUNK, D)
    for k in range(ROWS_PER_TILE // ZCHUNK):
      pltpu.sync_copy(
          zrow_v, acc_sh.at[pl.ds(s * ROWS_PER_TILE + k * ZCHUNK, ZCHUNK)]
      )
    plsc.subcore_barrier()

    base = wid * E_PER_TILE

    def i_start(i, b):
      off = base + i * CHUNK
      pltpu.async_copy(src_hbm.at[pl.ds(off, CHUNK)], srcv[b], isem[b])
      pltpu.async_copy(dst_hbm.at[pl.ds(off, CHUNK)], dstv[b], isem[b])

    def iw_gs(i, b):
      off = base + i * CHUNK
      pltpu.make_async_copy(src_hbm.at[pl.ds(off, CHUNK)], srcv[b], isem[b]).wait()
      pltpu.make_async_copy(dst_hbm.at[pl.ds(off, CHUNK)], dstv[b], isem[b]).wait()
      pltpu.async_copy(y_hbm.at[srcv[b]], rows[b], gsem[b])

    def gw_ss(i, b):
      pltpu.make_async_copy(y_hbm.at[srcv[b]], rows[b], gsem[b]).wait()
      pltpu.async_copy(rows[b], acc_sh.at[dstv[b]], ssem[b], add=True)

    def s_wait(i, b):
      pltpu.make_async_copy(rows[b], acc_sh.at[dstv[b]], ssem[b]).wait()

    def full_step(t, b):
      s_wait(t - NBUF, b)
      i_start(t, b)
      iw_gs(t - 1, (b + NBUF - 1) % NBUF)
      gw_ss(t - 3, (b + NBUF - 3) % NBUF)

    for t in range(NBUF):
      i_start(t, t % NBUF)
      if t - 1 >= 0:
        iw_gs(t - 1, (t - 1) % NBUF)
      if t - 3 >= 0:
        gw_ss(t - 3, (t - 3) % NBUF)

    def batch(j, carry):
      for b in range(NBUF):
        full_step(j * NBUF + b, b)
      return carry

    lax.fori_loop(1, STEPS // NBUF, batch, 0)

    for t in range(STEPS, STEPS + NBUF):
      s_wait(t - NBUF, (t - NBUF) % NBUF)
      if t - 1 < STEPS:
        iw_gs(t - 1, (t - 1) % NBUF)
      if t - 3 < STEPS:
        gw_ss(t - 3, (t - 3) % NBUF)

    plsc.subcore_barrier()

    pltpu.sync_copy(
        acc_sh.at[pl.ds(s * ROWS_PER_TILE, ROWS_PER_TILE)],
        out_hbm.at[c, pl.ds(s * ROWS_PER_TILE, ROWS_PER_TILE)],
    )

  return agg_kernel


_deg_call = _make_deg()
_agg_call = _make_agg()


R_BLK = 2000
GRID = N // R_BLK


def _tc1_body(x_ref, w_ref, dis_ref, xw_ref, y_ref):
  xw = jnp.dot(x_ref[...], w_ref[...], preferred_element_type=jnp.float32)
  xw_ref[...] = xw
  y_ref[...] = xw * dis_ref[...]


def _tc1(x, w1, dis2d):
  return pl.pallas_call(
      _tc1_body,
      grid=(GRID,),
      in_specs=[
          pl.BlockSpec((R_BLK, D), lambda i: (i, 0)),
          pl.BlockSpec((D, D), lambda i: (0, 0)),
          pl.BlockSpec((R_BLK, 1), lambda i: (i, 0)),
      ],
      out_specs=[
          pl.BlockSpec((R_BLK, D), lambda i: (i, 0)),
          pl.BlockSpec((R_BLK, D), lambda i: (i, 0)),
      ],
      out_shape=[
          jax.ShapeDtypeStruct((N, D), jnp.float32),
          jax.ShapeDtypeStruct((N, D), jnp.float32),
      ],
  )(x, w1, dis2d)


def _tc2_body(p_ref, xw_ref, dis_ref, b_ref, w_ref, xw2_ref, y2_ref):
  dis = dis_ref[...]
  agg = p_ref[0] + p_ref[1]
  h = jnp.maximum(dis * agg + (dis * dis) * xw_ref[...] + b_ref[...], 0.0)
  xw2 = jnp.dot(h, w_ref[...], preferred_element_type=jnp.float32)
  xw2_ref[...] = xw2
  y2_ref[...] = xw2 * dis


def _tc2(p, xw1, dis2d, b1, w2):
  return pl.pallas_call(
      _tc2_body,
      grid=(GRID,),
      in_specs=[
          pl.BlockSpec((NC, R_BLK, D), lambda i: (0, i, 0)),
          pl.BlockSpec((R_BLK, D), lambda i: (i, 0)),
          pl.BlockSpec((R_BLK, 1), lambda i: (i, 0)),
          pl.BlockSpec((1, D), lambda i: (0, 0)),
          pl.BlockSpec((D, D), lambda i: (0, 0)),
      ],
      out_specs=[
          pl.BlockSpec((R_BLK, D), lambda i: (i, 0)),
          pl.BlockSpec((R_BLK, D), lambda i: (i, 0)),
      ],
      out_shape=[
          jax.ShapeDtypeStruct((N, D), jnp.float32),
          jax.ShapeDtypeStruct((N, D), jnp.float32),
      ],
  )(p, xw1, dis2d, b1, w2)


def _tc3_body(p_ref, xw_ref, dis_ref, b_ref, wout_ref, bout_ref, out_ref):
  dis = dis_ref[...]
  agg = p_ref[0] + p_ref[1]
  h = jnp.maximum(dis * agg + (dis * dis) * xw_ref[...] + b_ref[...], 0.0)
  logit = jnp.dot(h, wout_ref[...], preferred_element_type=jnp.float32)
  logit = logit + bout_ref[...]
  out_ref[...] = 1.0 / (1.0 + jnp.exp(-logit))


def _tc3(p, xw2, dis2d, b2, w_out, b_out):
  return pl.pallas_call(
      _tc3_body,
      grid=(GRID,),
      in_specs=[
          pl.BlockSpec((NC, R_BLK, D), lambda i: (0, i, 0)),
          pl.BlockSpec((R_BLK, D), lambda i: (i, 0)),
          pl.BlockSpec((R_BLK, 1), lambda i: (i, 0)),
          pl.BlockSpec((1, D), lambda i: (0, 0)),
          pl.BlockSpec((D, 1), lambda i: (0, 0)),
          pl.BlockSpec((1, 1), lambda i: (0, 0)),
      ],
      out_specs=pl.BlockSpec((R_BLK, 1), lambda i: (i, 0)),
      out_shape=jax.ShapeDtypeStruct((N, 1), jnp.float32),
  )(p, xw2, dis2d, b2, w_out, b_out)


def kernel(x, edge_index, W1, b1, W2, b2, W_out, b_out):
  src = edge_index[0].astype(jnp.int32)
  dst = edge_index[1].astype(jnp.int32)

  deg_p = _deg_call(dst)
  deg = 1.0 + deg_p[0, :N, 0] + deg_p[1, :N, 0]
  dis2d = lax.rsqrt(deg)[:, None]

  xw1, y1 = _tc1(x, W1, dis2d)
  p1 = _agg_call(y1, src, dst)
  xw2, y2 = _tc2(p1, xw1, dis2d, b1.reshape(1, D), W2)
  p2 = _agg_call(y2, src, dst)
  out = _tc3(p2, xw2, dis2d, b2.reshape(1, D), W_out, b_out.reshape(1, 1))
  return out[:, 0]


# speedup vs baseline: 29.6185x; 1.0742x over previous
import functools

import jax
import jax.numpy as jnp
from jax import lax
from jax.experimental import pallas as pl
from jax.experimental.pallas import tpu as pltpu
from jax.experimental.pallas import tpu_sc as plsc

N = 10000
E = 320000
D = 128

NC = 2
NS = 16
NW = NC * NS

E_PER_TILE = E // NW
CHUNK = 80
STEPS = E_PER_TILE // CHUNK

N_ROWS = 10240
ROWS_PER_TILE = N_ROWS // NS
ZCHUNK = 32

_MESH = plsc.VectorSubcoreMesh(core_axis_name="c", subcore_axis_name="s")


def _zero_vmem_2d(ref, rows, cols):
  zero = jnp.zeros((16,), jnp.float32)

  def body(r, carry):
    for j in range(cols // 16):
      ref[r, pl.ds(j * 16, 16)] = zero
    return carry

  lax.fori_loop(0, rows, body, 0)


DEG_CHUNK = 80
DEG_STEPS = E_PER_TILE // DEG_CHUNK
DEG_NBUF = 5


def _make_deg():
  scratch = (
      [pltpu.VMEM((DEG_CHUNK,), jnp.int32) for _ in range(DEG_NBUF)]
      + [
          pltpu.VMEM((DEG_CHUNK, D), jnp.float32),
          pltpu.VMEM((ZCHUNK, D), jnp.float32),
          pltpu.VMEM_SHARED((N_ROWS, D), jnp.float32),
      ]
      + [pltpu.SemaphoreType.DMA for _ in range(2 * DEG_NBUF)]
  )

  @functools.partial(
      pl.kernel,
      out_type=jax.ShapeDtypeStruct((NC, N_ROWS, D), jnp.float32),
      mesh=_MESH,
      scratch_types=scratch,
  )
  def deg_kernel(dst_hbm, out_hbm, *refs):
    dstv = refs[0:DEG_NBUF]
    ones_v = refs[DEG_NBUF]
    zrow_v = refs[DEG_NBUF + 1]
    acc_sh = refs[DEG_NBUF + 2]
    isem = refs[DEG_NBUF + 3:DEG_NBUF + 3 + DEG_NBUF]
    ssem = refs[DEG_NBUF + 3 + DEG_NBUF:DEG_NBUF + 3 + 2 * DEG_NBUF]

    c = lax.axis_index("c")
    s = lax.axis_index("s")
    wid = s * NC + c

    one = jnp.full((16,), 1.0, jnp.float32)

    def fill(r, carry):
      for j in range(D // 16):
        ones_v[r, pl.ds(j * 16, 16)] = one
      return carry

    lax.fori_loop(0, DEG_CHUNK, fill, 0)
    _zero_vmem_2d(zrow_v, ZCHUNK, D)
    for k in range(ROWS_PER_TILE // ZCHUNK):
      pltpu.sync_copy(
          zrow_v, acc_sh.at[pl.ds(s * ROWS_PER_TILE + k * ZCHUNK, ZCHUNK)]
      )
    plsc.subcore_barrier()

    base = wid * E_PER_TILE

    def i_start(i, b):
      pltpu.async_copy(
          dst_hbm.at[pl.ds(base + i * DEG_CHUNK, DEG_CHUNK)], dstv[b], isem[b]
      )

    def iw_ss(i, b):
      pltpu.make_async_copy(
          dst_hbm.at[pl.ds(base + i * DEG_CHUNK, DEG_CHUNK)], dstv[b], isem[b]
      ).wait()
      pltpu.async_copy(ones_v, acc_sh.at[dstv[b]], ssem[b], add=True)

    def s_wait(b):
      pltpu.make_async_copy(ones_v, acc_sh.at[dstv[b]], ssem[b]).wait()

    for t in range(DEG_NBUF):
      i_start(t, t)
      if t >= 1:
        iw_ss(t - 1, t - 1)

    def batch(j, carry):
      for b in range(DEG_NBUF):
        t = j * DEG_NBUF + b
        s_wait(b)
        i_start(t, b)
        iw_ss(t - 1, (b + DEG_NBUF - 1) % DEG_NBUF)
      return carry

    lax.fori_loop(1, DEG_STEPS // DEG_NBUF, batch, 0)

    for t in range(DEG_STEPS, DEG_STEPS + DEG_NBUF):
      s_wait((t - DEG_NBUF) % DEG_NBUF)
      if t - 1 < DEG_STEPS:
        iw_ss(t - 1, (t - 1) % DEG_NBUF)

    plsc.subcore_barrier()

    pltpu.sync_copy(
        acc_sh.at[pl.ds(s * ROWS_PER_TILE, ROWS_PER_TILE)],
        out_hbm.at[c, pl.ds(s * ROWS_PER_TILE, ROWS_PER_TILE)],
    )

  return deg_kernel


NBUF = 4


def _make_agg():
  scratch = (
      [pltpu.VMEM((CHUNK,), jnp.int32) for _ in range(NBUF)]
      + [pltpu.VMEM((CHUNK,), jnp.int32) for _ in range(NBUF)]
      + [pltpu.VMEM((CHUNK, D), jnp.float32) for _ in range(NBUF)]
      + [pltpu.VMEM((ZCHUNK, D), jnp.float32)]
      + [pltpu.VMEM_SHARED((N_ROWS, D), jnp.float32)]
      + [pltpu.SemaphoreType.DMA for _ in range(3 * NBUF)]
  )

  @functools.partial(
      pl.kernel,
      out_type=jax.ShapeDtypeStruct((NC, N_ROWS, D), jnp.float32),
      mesh=_MESH,
      scratch_types=scratch,
  )
  def agg_kernel(y_hbm, src_hbm, dst_hbm, out_hbm, *refs):
    srcv = refs[0:NBUF]
    dstv = refs[NBUF:2 * NBUF]
    rows = refs[2 * NBUF:3 * NBUF]
    zrow_v = refs[3 * NBUF]
    acc_sh = refs[3 * NBUF + 1]
    isem = refs[3 * NBUF + 2:3 * NBUF + 2 + NBUF]
    gsem = refs[3 * NBUF + 2 + NBUF:3 * NBUF + 2 + 2 * NBUF]
    ssem = refs[3 * NBUF + 2 + 2 * NBUF:3 * NBUF + 2 + 3 * NBUF]

    c = lax.axis_index("c")
    s = lax.axis_index("s")
    wid = s * NC + c

    _zero_vmem_2d(zrow_v, ZCHUNK, D)
    for k in range(ROWS_PER_TILE // ZCHUNK):
      pltpu.sync_copy(
          zrow_v, acc_sh.at[pl.ds(s * ROWS_PER_TILE + k * ZCHUNK, ZCHUNK)]
      )
    plsc.subcore_barrier()

    base = wid * E_PER_TILE

    def i_start(i, b):
      off = base + i * CHUNK
      pltpu.async_copy(src_hbm.at[pl.ds(off, CHUNK)], srcv[b], isem[b])
      pltpu.async_copy(dst_hbm.at[pl.ds(off, CHUNK)], dstv[b], isem[b])

    def iw_gs(i, b):
      off = base + i * CHUNK
      pltpu.make_async_copy(src_hbm.at[pl.ds(off, CHUNK)], srcv[b], isem[b]).wait()
      pltpu.make_async_copy(dst_hbm.at[pl.ds(off, CHUNK)], dstv[b], isem[b]).wait()
      pltpu.async_copy(y_hbm.at[srcv[b]], rows[b], gsem[b])

    def gw_ss(i, b):
      pltpu.make_async_copy(y_hbm.at[srcv[b]], rows[b], gsem[b]).wait()
      pltpu.async_copy(rows[b], acc_sh.at[dstv[b]], ssem[b], add=True)

    def s_wait(i, b):
      pltpu.make_async_copy(rows[b], acc_sh.at[dstv[b]], ssem[b]).wait()

    def full_step(t, b):
      s_wait(t - NBUF, b)
      i_start(t, b)
      iw_gs(t - 1, (b + NBUF - 1) % NBUF)
      gw_ss(t - 2, (b + NBUF - 2) % NBUF)

    for t in range(NBUF):
      i_start(t, t % NBUF)
      if t - 1 >= 0:
        iw_gs(t - 1, (t - 1) % NBUF)
      if t - 2 >= 0:
        gw_ss(t - 2, (t - 2) % NBUF)

    def batch(j, carry):
      for b in range(NBUF):
        full_step(j * NBUF + b, b)
      return carry

    lax.fori_loop(1, STEPS // NBUF, batch, 0)

    for t in range((STEPS // NBUF) * NBUF, STEPS):
      full_step(t, t % NBUF)
    for t in range(STEPS, STEPS + NBUF):
      s_wait(t - NBUF, (t - NBUF) % NBUF)
      if t - 1 < STEPS:
        iw_gs(t - 1, (t - 1) % NBUF)
      if t - 2 < STEPS:
        gw_ss(t - 2, (t - 2) % NBUF)

    plsc.subcore_barrier()

    pltpu.sync_copy(
        acc_sh.at[pl.ds(s * ROWS_PER_TILE, ROWS_PER_TILE)],
        out_hbm.at[c, pl.ds(s * ROWS_PER_TILE, ROWS_PER_TILE)],
    )

  return agg_kernel


_deg_call = _make_deg()
_agg_call = _make_agg()


R_BLK = 2000
GRID = N // R_BLK


def _tc1_body(x_ref, w_ref, dis_ref, xw_ref, y_ref):
  xw = jnp.dot(x_ref[...], w_ref[...], preferred_element_type=jnp.float32)
  xw_ref[...] = xw
  y_ref[...] = xw * dis_ref[...]


def _tc1(x, w1, dis2d):
  return pl.pallas_call(
      _tc1_body,
      grid=(GRID,),
      in_specs=[
          pl.BlockSpec((R_BLK, D), lambda i: (i, 0)),
          pl.BlockSpec((D, D), lambda i: (0, 0)),
          pl.BlockSpec((R_BLK, 1), lambda i: (i, 0)),
      ],
      out_specs=[
          pl.BlockSpec((R_BLK, D), lambda i: (i, 0)),
          pl.BlockSpec((R_BLK, D), lambda i: (i, 0)),
      ],
      out_shape=[
          jax.ShapeDtypeStruct((N, D), jnp.float32),
          jax.ShapeDtypeStruct((N, D), jnp.float32),
      ],
  )(x, w1, dis2d)


def _tc2_body(p_ref, xw_ref, dis_ref, b_ref, w_ref, xw2_ref, y2_ref):
  dis = dis_ref[...]
  agg = p_ref[0] + p_ref[1]
  h = jnp.maximum(dis * agg + (dis * dis) * xw_ref[...] + b_ref[...], 0.0)
  xw2 = jnp.dot(h, w_ref[...], preferred_element_type=jnp.float32)
  xw2_ref[...] = xw2
  y2_ref[...] = xw2 * dis


def _tc2(p, xw1, dis2d, b1, w2):
  return pl.pallas_call(
      _tc2_body,
      grid=(GRID,),
      in_specs=[
          pl.BlockSpec((NC, R_BLK, D), lambda i: (0, i, 0)),
          pl.BlockSpec((R_BLK, D), lambda i: (i, 0)),
          pl.BlockSpec((R_BLK, 1), lambda i: (i, 0)),
          pl.BlockSpec((1, D), lambda i: (0, 0)),
          pl.BlockSpec((D, D), lambda i: (0, 0)),
      ],
      out_specs=[
          pl.BlockSpec((R_BLK, D), lambda i: (i, 0)),
          pl.BlockSpec((R_BLK, D), lambda i: (i, 0)),
      ],
      out_shape=[
          jax.ShapeDtypeStruct((N, D), jnp.float32),
          jax.ShapeDtypeStruct((N, D), jnp.float32),
      ],
  )(p, xw1, dis2d, b1, w2)


def _tc3_body(p_ref, xw_ref, dis_ref, b_ref, wout_ref, bout_ref, out_ref):
  dis = dis_ref[...]
  agg = p_ref[0] + p_ref[1]
  h = jnp.maximum(dis * agg + (dis * dis) * xw_ref[...] + b_ref[...], 0.0)
  logit = jnp.dot(h, wout_ref[...], preferred_element_type=jnp.float32)
  logit = logit + bout_ref[...]
  out_ref[...] = 1.0 / (1.0 + jnp.exp(-logit))


def _tc3(p, xw2, dis2d, b2, w_out, b_out):
  return pl.pallas_call(
      _tc3_body,
      grid=(GRID,),
      in_specs=[
          pl.BlockSpec((NC, R_BLK, D), lambda i: (0, i, 0)),
          pl.BlockSpec((R_BLK, D), lambda i: (i, 0)),
          pl.BlockSpec((R_BLK, 1), lambda i: (i, 0)),
          pl.BlockSpec((1, D), lambda i: (0, 0)),
          pl.BlockSpec((D, 1), lambda i: (0, 0)),
          pl.BlockSpec((1, 1), lambda i: (0, 0)),
      ],
      out_specs=pl.BlockSpec((R_BLK, 1), lambda i: (i, 0)),
      out_shape=jax.ShapeDtypeStruct((N, 1), jnp.float32),
  )(p, xw2, dis2d, b2, w_out, b_out)


def kernel(x, edge_index, W1, b1, W2, b2, W_out, b_out):
  src = edge_index[0].astype(jnp.int32)
  dst = edge_index[1].astype(jnp.int32)

  deg_p = _deg_call(dst)
  deg = 1.0 + deg_p[0, :N, 0] + deg_p[1, :N, 0]
  dis2d = lax.rsqrt(deg)[:, None]

  xw1, y1 = _tc1(x, W1, dis2d)
  p1 = _agg_call(y1, src, dst)
  xw2, y2 = _tc2(p1, xw1, dis2d, b1.reshape(1, D), W2)
  p2 = _agg_call(y2, src, dst)
  out = _tc3(p2, xw2, dis2d, b2.reshape(1, D), W_out, b_out.reshape(1, 1))
  return out[:, 0]


# speedup vs baseline: 29.6742x; 1.0019x over previous
import functools

import jax
import jax.numpy as jnp
from jax import lax
from jax.experimental import pallas as pl
from jax.experimental.pallas import tpu as pltpu
from jax.experimental.pallas import tpu_sc as plsc

N = 10000
E = 320000
D = 128

NC = 2
NS = 16
NW = NC * NS

E_PER_TILE = E // NW
CHUNK = 80
STEPS = E_PER_TILE // CHUNK

N_ROWS = 10240
ROWS_PER_TILE = N_ROWS // NS
ZCHUNK = 32

_MESH = plsc.VectorSubcoreMesh(core_axis_name="c", subcore_axis_name="s")


def _zero_vmem_2d(ref, rows, cols):
  zero = jnp.zeros((16,), jnp.float32)

  def body(r, carry):
    for j in range(cols // 16):
      ref[r, pl.ds(j * 16, 16)] = zero
    return carry

  lax.fori_loop(0, rows, body, 0)


DEG_CHUNK = 80
DEG_STEPS = E_PER_TILE // DEG_CHUNK
DEG_NBUF = 5


def _make_deg():
  scratch = (
      [pltpu.VMEM((DEG_CHUNK,), jnp.int32) for _ in range(DEG_NBUF)]
      + [
          pltpu.VMEM((DEG_CHUNK, D), jnp.float32),
          pltpu.VMEM((ZCHUNK, D), jnp.float32),
          pltpu.VMEM_SHARED((N_ROWS, D), jnp.float32),
      ]
      + [pltpu.SemaphoreType.DMA for _ in range(2 * DEG_NBUF)]
  )

  @functools.partial(
      pl.kernel,
      out_type=jax.ShapeDtypeStruct((NC, N_ROWS, D), jnp.float32),
      mesh=_MESH,
      scratch_types=scratch,
  )
  def deg_kernel(dst_hbm, out_hbm, *refs):
    dstv = refs[0:DEG_NBUF]
    ones_v = refs[DEG_NBUF]
    zrow_v = refs[DEG_NBUF + 1]
    acc_sh = refs[DEG_NBUF + 2]
    isem = refs[DEG_NBUF + 3:DEG_NBUF + 3 + DEG_NBUF]
    ssem = refs[DEG_NBUF + 3 + DEG_NBUF:DEG_NBUF + 3 + 2 * DEG_NBUF]

    c = lax.axis_index("c")
    s = lax.axis_index("s")
    wid = s * NC + c

    one = jnp.full((16,), 1.0, jnp.float32)

    def fill(r, carry):
      for j in range(D // 16):
        ones_v[r, pl.ds(j * 16, 16)] = one
      return carry

    lax.fori_loop(0, DEG_CHUNK, fill, 0)
    _zero_vmem_2d(zrow_v, ZCHUNK, D)
    for k in range(ROWS_PER_TILE // ZCHUNK):
      pltpu.sync_copy(
          zrow_v, acc_sh.at[pl.ds(s * ROWS_PER_TILE + k * ZCHUNK, ZCHUNK)]
      )
    plsc.subcore_barrier()

    base = wid * E_PER_TILE

    def i_start(i, b):
      pltpu.async_copy(
          dst_hbm.at[pl.ds(base + i * DEG_CHUNK, DEG_CHUNK)], dstv[b], isem[b]
      )

    def iw_ss(i, b):
      pltpu.make_async_copy(
          dst_hbm.at[pl.ds(base + i * DEG_CHUNK, DEG_CHUNK)], dstv[b], isem[b]
      ).wait()
      pltpu.async_copy(ones_v, acc_sh.at[dstv[b]], ssem[b], add=True)

    def s_wait(b):
      pltpu.make_async_copy(ones_v, acc_sh.at[dstv[b]], ssem[b]).wait()

    for t in range(DEG_NBUF):
      i_start(t, t)
      if t >= 1:
        iw_ss(t - 1, t - 1)

    def batch(j, carry):
      for b in range(DEG_NBUF):
        t = j * DEG_NBUF + b
        s_wait(b)
        i_start(t, b)
        iw_ss(t - 1, (b + DEG_NBUF - 1) % DEG_NBUF)
      return carry

    lax.fori_loop(1, DEG_STEPS // DEG_NBUF, batch, 0)

    for t in range(DEG_STEPS, DEG_STEPS + DEG_NBUF):
      s_wait((t - DEG_NBUF) % DEG_NBUF)
      if t - 1 < DEG_STEPS:
        iw_ss(t - 1, (t - 1) % DEG_NBUF)

    plsc.subcore_barrier()

    pltpu.sync_copy(
        acc_sh.at[pl.ds(s * ROWS_PER_TILE, ROWS_PER_TILE)],
        out_hbm.at[c, pl.ds(s * ROWS_PER_TILE, ROWS_PER_TILE)],
    )

  return deg_kernel


NBUF = 4


def _make_agg():
  scratch = (
      [pltpu.VMEM((CHUNK,), jnp.int32) for _ in range(NBUF)]
      + [pltpu.VMEM((CHUNK,), jnp.int32) for _ in range(NBUF)]
      + [pltpu.VMEM((CHUNK, D), jnp.float32) for _ in range(NBUF)]
      + [pltpu.VMEM((ZCHUNK, D), jnp.float32)]
      + [pltpu.VMEM_SHARED((N_ROWS, D), jnp.float32)]
      + [pltpu.SemaphoreType.DMA for _ in range(3 * NBUF)]
  )

  @functools.partial(
      pl.kernel,
      out_type=jax.ShapeDtypeStruct((NC, N_ROWS, D), jnp.float32),
      mesh=_MESH,
      scratch_types=scratch,
  )
  def agg_kernel(y_hbm, src_hbm, dst_hbm, out_hbm, *refs):
    srcv = refs[0:NBUF]
    dstv = refs[NBUF:2 * NBUF]
    rows = refs[2 * NBUF:3 * NBUF]
    zrow_v = refs[3 * NBUF]
    acc_sh = refs[3 * NBUF + 1]
    isem = refs[3 * NBUF + 2:3 * NBUF + 2 + NBUF]
    gsem = refs[3 * NBUF + 2 + NBUF:3 * NBUF + 2 + 2 * NBUF]
    ssem = refs[3 * NBUF + 2 + 2 * NBUF:3 * NBUF + 2 + 3 * NBUF]

    c = lax.axis_index("c")
    s = lax.axis_index("s")
    wid = s * NC + c

    _zero_vmem_2d(zrow_v, ZCHUNK, D)
    for k in range(ROWS_PER_TILE // ZCHUNK):
      pltpu.sync_copy(
          zrow_v, acc_sh.at[pl.ds(s * ROWS_PER_TILE + k * ZCHUNK, ZCHUNK)]
      )
    plsc.subcore_barrier()

    base = wid * E_PER_TILE

    def i_start(i, b):
      off = base + i * CHUNK
      pltpu.async_copy(src_hbm.at[pl.ds(off, CHUNK)], srcv[b], isem[b])
      pltpu.async_copy(dst_hbm.at[pl.ds(off, CHUNK)], dstv[b], isem[b])

    def iw_gs(i, b):
      off = base + i * CHUNK
      pltpu.make_async_copy(src_hbm.at[pl.ds(off, CHUNK)], srcv[b], isem[b]).wait()
      pltpu.make_async_copy(dst_hbm.at[pl.ds(off, CHUNK)], dstv[b], isem[b]).wait()
      pltpu.async_copy(y_hbm.at[srcv[b]], rows[b], gsem[b])

    def gw_ss(i, b):
      pltpu.make_async_copy(y_hbm.at[srcv[b]], rows[b], gsem[b]).wait()
      pltpu.async_copy(rows[b], acc_sh.at[dstv[b]], ssem[b], add=True)

    def s_wait(i, b):
      pltpu.make_async_copy(rows[b], acc_sh.at[dstv[b]], ssem[b]).wait()

    def full_step(t, b):
      s_wait(t - NBUF, b)
      i_start(t, b)
      iw_gs(t - 1, (b + NBUF - 1) % NBUF)
      gw_ss(t - 2, (b + NBUF - 2) % NBUF)

    for t in range(NBUF):
      i_start(t, t % NBUF)
      if t - 1 >= 0:
        iw_gs(t - 1, (t - 1) % NBUF)
      if t - 2 >= 0:
        gw_ss(t - 2, (t - 2) % NBUF)

    def batch(j, carry):
      for b in range(NBUF):
        full_step(j * NBUF + b, b)
      return carry

    lax.fori_loop(1, STEPS // NBUF, batch, 0)

    for t in range((STEPS // NBUF) * NBUF, STEPS):
      full_step(t, t % NBUF)
    for t in range(STEPS, STEPS + NBUF):
      s_wait(t - NBUF, (t - NBUF) % NBUF)
      if t - 1 < STEPS:
        iw_gs(t - 1, (t - 1) % NBUF)
      if t - 2 < STEPS:
        gw_ss(t - 2, (t - 2) % NBUF)

    plsc.subcore_barrier()

    pltpu.sync_copy(
        acc_sh.at[pl.ds(s * ROWS_PER_TILE, ROWS_PER_TILE)],
        out_hbm.at[c, pl.ds(s * ROWS_PER_TILE, ROWS_PER_TILE)],
    )

  return agg_kernel


_deg_call = _make_deg()
_agg_call = _make_agg()


R_BLK = 2000
GRID = N // R_BLK


def _tc_mm_body(x_ref, w_ref, xw_ref):
  xw_ref[...] = jnp.dot(
      x_ref[...], w_ref[...], preferred_element_type=jnp.float32
  )


def _tc_mm(x, w1):
  return pl.pallas_call(
      _tc_mm_body,
      grid=(GRID,),
      in_specs=[
          pl.BlockSpec((R_BLK, D), lambda i: (i, 0)),
          pl.BlockSpec((D, D), lambda i: (0, 0)),
      ],
      out_specs=pl.BlockSpec((R_BLK, D), lambda i: (i, 0)),
      out_shape=jax.ShapeDtypeStruct((N, D), jnp.float32),
  )(x, w1)


def _tc_scale_body(xw_ref, dis_ref, y_ref):
  y_ref[...] = xw_ref[...] * dis_ref[...]


def _tc_scale(xw, dis2d):
  return pl.pallas_call(
      _tc_scale_body,
      grid=(GRID,),
      in_specs=[
          pl.BlockSpec((R_BLK, D), lambda i: (i, 0)),
          pl.BlockSpec((R_BLK, 1), lambda i: (i, 0)),
      ],
      out_specs=pl.BlockSpec((R_BLK, D), lambda i: (i, 0)),
      out_shape=jax.ShapeDtypeStruct((N, D), jnp.float32),
  )(xw, dis2d)


def _tc2_body(p_ref, xw_ref, dis_ref, b_ref, w_ref, xw2_ref, y2_ref):
  dis = dis_ref[...]
  agg = p_ref[0] + p_ref[1]
  h = jnp.maximum(dis * agg + (dis * dis) * xw_ref[...] + b_ref[...], 0.0)
  xw2 = jnp.dot(h, w_ref[...], preferred_element_type=jnp.float32)
  xw2_ref[...] = xw2
  y2_ref[...] = xw2 * dis


def _tc2(p, xw1, dis2d, b1, w2):
  return pl.pallas_call(
      _tc2_body,
      grid=(GRID,),
      in_specs=[
          pl.BlockSpec((NC, R_BLK, D), lambda i: (0, i, 0)),
          pl.BlockSpec((R_BLK, D), lambda i: (i, 0)),
          pl.BlockSpec((R_BLK, 1), lambda i: (i, 0)),
          pl.BlockSpec((1, D), lambda i: (0, 0)),
          pl.BlockSpec((D, D), lambda i: (0, 0)),
      ],
      out_specs=[
          pl.BlockSpec((R_BLK, D), lambda i: (i, 0)),
          pl.BlockSpec((R_BLK, D), lambda i: (i, 0)),
      ],
      out_shape=[
          jax.ShapeDtypeStruct((N, D), jnp.float32),
          jax.ShapeDtypeStruct((N, D), jnp.float32),
      ],
  )(p, xw1, dis2d, b1, w2)


def _tc3_body(p_ref, xw_ref, dis_ref, b_ref, wout_ref, bout_ref, out_ref):
  dis = dis_ref[...]
  agg = p_ref[0] + p_ref[1]
  h = jnp.maximum(dis * agg + (dis * dis) * xw_ref[...] + b_ref[...], 0.0)
  logit = jnp.dot(h, wout_ref[...], preferred_element_type=jnp.float32)
  logit = logit + bout_ref[...]
  out_ref[...] = 1.0 / (1.0 + jnp.exp(-logit))


def _tc3(p, xw2, dis2d, b2, w_out, b_out):
  return pl.pallas_call(
      _tc3_body,
      grid=(GRID,),
      in_specs=[
          pl.BlockSpec((NC, R_BLK, D), lambda i: (0, i, 0)),
          pl.BlockSpec((R_BLK, D), lambda i: (i, 0)),
          pl.BlockSpec((R_BLK, 1), lambda i: (i, 0)),
          pl.BlockSpec((1, D), lambda i: (0, 0)),
          pl.BlockSpec((D, 1), lambda i: (0, 0)),
          pl.BlockSpec((1, 1), lambda i: (0, 0)),
      ],
      out_specs=pl.BlockSpec((R_BLK, 1), lambda i: (i, 0)),
      out_shape=jax.ShapeDtypeStruct((N, 1), jnp.float32),
  )(p, xw2, dis2d, b2, w_out, b_out)


def kernel(x, edge_index, W1, b1, W2, b2, W_out, b_out):
  src = edge_index[0].astype(jnp.int32)
  dst = edge_index[1].astype(jnp.int32)

  deg_p = _deg_call(dst)
  xw1 = _tc_mm(x, W1)
  deg = 1.0 + deg_p[0, :N, 0] + deg_p[1, :N, 0]
  dis2d = lax.rsqrt(deg)[:, None]

  y1 = _tc_scale(xw1, dis2d)
  p1 = _agg_call(y1, src, dst)
  xw2, y2 = _tc2(p1, xw1, dis2d, b1.reshape(1, D), W2)
  p2 = _agg_call(y2, src, dst)
  out = _tc3(p2, xw2, dis2d, b2.reshape(1, D), W_out, b_out.reshape(1, 1))
  return out[:, 0]


# speedup vs baseline: 29.9454x; 1.0091x over previous
import functools

import jax
import jax.numpy as jnp
from jax import lax
from jax.experimental import pallas as pl
from jax.experimental.pallas import tpu as pltpu
from jax.experimental.pallas import tpu_sc as plsc

N = 10000
E = 320000
D = 128

NC = 2
NS = 16
NW = NC * NS

E_PER_TILE = E // NW
CHUNK = 80
STEPS = E_PER_TILE // CHUNK

N_ROWS = 10240
ROWS_PER_TILE = N_ROWS // NS
ZCHUNK = 32

_MESH = plsc.VectorSubcoreMesh(core_axis_name="c", subcore_axis_name="s")


def _zero_vmem_2d(ref, rows, cols):
  zero = jnp.zeros((16,), jnp.float32)

  def body(r, carry):
    for j in range(cols // 16):
      ref[r, pl.ds(j * 16, 16)] = zero
    return carry

  lax.fori_loop(0, rows, body, 0)


def _zero_acc(zrow_v, zch, acc_sh, row_base, nrows, zsem):
  ncopies = nrows // zch
  k = 0
  while k < ncopies:
    batchn = min(5, ncopies - k)
    for t in range(batchn):
      pltpu.async_copy(
          zrow_v, acc_sh.at[pl.ds(row_base + (k + t) * zch, zch)], zsem
      )
    for t in range(batchn):
      pltpu.make_async_copy(
          zrow_v, acc_sh.at[pl.ds(row_base, zch)], zsem
      ).wait()
    k += batchn


DEG_CHUNK = 80
DEG_STEPS = E_PER_TILE // DEG_CHUNK
DEG_NBUF = 5


def _make_deg():
  scratch = (
      [pltpu.VMEM((DEG_CHUNK,), jnp.int32) for _ in range(DEG_NBUF)]
      + [
          pltpu.VMEM((DEG_CHUNK, D), jnp.float32),
          pltpu.VMEM((128, D), jnp.float32),
          pltpu.VMEM_SHARED((N_ROWS, D), jnp.float32),
      ]
      + [pltpu.SemaphoreType.DMA for _ in range(2 * DEG_NBUF + 1)]
  )

  @functools.partial(
      pl.kernel,
      out_type=jax.ShapeDtypeStruct((NC, N_ROWS, D), jnp.float32),
      mesh=_MESH,
      scratch_types=scratch,
  )
  def deg_kernel(dst_hbm, out_hbm, *refs):
    dstv = refs[0:DEG_NBUF]
    ones_v = refs[DEG_NBUF]
    zrow_v = refs[DEG_NBUF + 1]
    acc_sh = refs[DEG_NBUF + 2]
    isem = refs[DEG_NBUF + 3:DEG_NBUF + 3 + DEG_NBUF]
    ssem = refs[DEG_NBUF + 3 + DEG_NBUF:DEG_NBUF + 3 + 2 * DEG_NBUF]
    zsem = refs[DEG_NBUF + 3 + 2 * DEG_NBUF]

    c = lax.axis_index("c")
    s = lax.axis_index("s")
    wid = s * NC + c

    one = jnp.full((16,), 1.0, jnp.float32)

    def fill(r, carry):
      for j in range(D // 16):
        ones_v[r, pl.ds(j * 16, 16)] = one
      return carry

    lax.fori_loop(0, DEG_CHUNK, fill, 0)
    _zero_vmem_2d(zrow_v, 128, D)
    _zero_acc(zrow_v, 128, acc_sh, s * ROWS_PER_TILE, ROWS_PER_TILE, zsem)
    plsc.subcore_barrier()

    base = wid * E_PER_TILE

    def i_start(i, b):
      pltpu.async_copy(
          dst_hbm.at[pl.ds(base + i * DEG_CHUNK, DEG_CHUNK)], dstv[b], isem[b]
      )

    def iw_ss(i, b):
      pltpu.make_async_copy(
          dst_hbm.at[pl.ds(base + i * DEG_CHUNK, DEG_CHUNK)], dstv[b], isem[b]
      ).wait()
      pltpu.async_copy(ones_v, acc_sh.at[dstv[b]], ssem[b], add=True)

    def s_wait(b):
      pltpu.make_async_copy(ones_v, acc_sh.at[dstv[b]], ssem[b]).wait()

    for t in range(DEG_NBUF):
      i_start(t, t)
      if t >= 1:
        iw_ss(t - 1, t - 1)

    def batch(j, carry):
      for b in range(DEG_NBUF):
        t = j * DEG_NBUF + b
        s_wait(b)
        i_start(t, b)
        iw_ss(t - 1, (b + DEG_NBUF - 1) % DEG_NBUF)
      return carry

    lax.fori_loop(1, DEG_STEPS // DEG_NBUF, batch, 0)

    for t in range(DEG_STEPS, DEG_STEPS + DEG_NBUF):
      s_wait((t - DEG_NBUF) % DEG_NBUF)
      if t - 1 < DEG_STEPS:
        iw_ss(t - 1, (t - 1) % DEG_NBUF)

    plsc.subcore_barrier()

    pltpu.sync_copy(
        acc_sh.at[pl.ds(s * ROWS_PER_TILE, ROWS_PER_TILE)],
        out_hbm.at[c, pl.ds(s * ROWS_PER_TILE, ROWS_PER_TILE)],
    )

  return deg_kernel


NBUF = 4


def _make_agg():
  scratch = (
      [pltpu.VMEM((CHUNK,), jnp.int32) for _ in range(NBUF)]
      + [pltpu.VMEM((CHUNK,), jnp.int32) for _ in range(NBUF)]
      + [pltpu.VMEM((CHUNK, D), jnp.float32) for _ in range(NBUF)]
      + [pltpu.VMEM((ZCHUNK, D), jnp.float32)]
      + [pltpu.VMEM_SHARED((N_ROWS, D), jnp.float32)]
      + [pltpu.SemaphoreType.DMA for _ in range(3 * NBUF + 1)]
  )

  @functools.partial(
      pl.kernel,
      out_type=jax.ShapeDtypeStruct((NC, N_ROWS, D), jnp.float32),
      mesh=_MESH,
      scratch_types=scratch,
  )
  def agg_kernel(y_hbm, src_hbm, dst_hbm, out_hbm, *refs):
    srcv = refs[0:NBUF]
    dstv = refs[NBUF:2 * NBUF]
    rows = refs[2 * NBUF:3 * NBUF]
    zrow_v = refs[3 * NBUF]
    acc_sh = refs[3 * NBUF + 1]
    isem = refs[3 * NBUF + 2:3 * NBUF + 2 + NBUF]
    gsem = refs[3 * NBUF + 2 + NBUF:3 * NBUF + 2 + 2 * NBUF]
    ssem = refs[3 * NBUF + 2 + 2 * NBUF:3 * NBUF + 2 + 3 * NBUF]
    zsem = refs[3 * NBUF + 2 + 3 * NBUF]

    c = lax.axis_index("c")
    s = lax.axis_index("s")
    wid = s * NC + c

    _zero_vmem_2d(zrow_v, ZCHUNK, D)
    _zero_acc(zrow_v, ZCHUNK, acc_sh, s * ROWS_PER_TILE, ROWS_PER_TILE, zsem)
    plsc.subcore_barrier()

    base = wid * E_PER_TILE

    def i_start(i, b):
      off = base + i * CHUNK
      pltpu.async_copy(src_hbm.at[pl.ds(off, CHUNK)], srcv[b], isem[b])
      pltpu.async_copy(dst_hbm.at[pl.ds(off, CHUNK)], dstv[b], isem[b])

    def iw_gs(i, b):
      off = base + i * CHUNK
      pltpu.make_async_copy(src_hbm.at[pl.ds(off, CHUNK)], srcv[b], isem[b]).wait()
      pltpu.make_async_copy(dst_hbm.at[pl.ds(off, CHUNK)], dstv[b], isem[b]).wait()
      pltpu.async_copy(y_hbm.at[srcv[b]], rows[b], gsem[b])

    def gw_ss(i, b):
      pltpu.make_async_copy(y_hbm.at[srcv[b]], rows[b], gsem[b]).wait()
      pltpu.async_copy(rows[b], acc_sh.at[dstv[b]], ssem[b], add=True)

    def s_wait(i, b):
      pltpu.make_async_copy(rows[b], acc_sh.at[dstv[b]], ssem[b]).wait()

    def full_step(t, b):
      s_wait(t - NBUF, b)
      i_start(t, b)
      iw_gs(t - 1, (b + NBUF - 1) % NBUF)
      gw_ss(t - 2, (b + NBUF - 2) % NBUF)

    for t in range(NBUF):
      i_start(t, t % NBUF)
      if t - 1 >= 0:
        iw_gs(t - 1, (t - 1) % NBUF)
      if t - 2 >= 0:
        gw_ss(t - 2, (t - 2) % NBUF)

    def batch(j, carry):
      for b in range(NBUF):
        full_step(j * NBUF + b, b)
      return carry

    lax.fori_loop(1, STEPS // NBUF, batch, 0)

    for t in range((STEPS // NBUF) * NBUF, STEPS):
      full_step(t, t % NBUF)
    for t in range(STEPS, STEPS + NBUF):
      s_wait(t - NBUF, (t - NBUF) % NBUF)
      if t - 1 < STEPS:
        iw_gs(t - 1, (t - 1) % NBUF)
      if t - 2 < STEPS:
        gw_ss(t - 2, (t - 2) % NBUF)

    plsc.subcore_barrier()

    pltpu.sync_copy(
        acc_sh.at[pl.ds(s * ROWS_PER_TILE, ROWS_PER_TILE)],
        out_hbm.at[c, pl.ds(s * ROWS_PER_TILE, ROWS_PER_TILE)],
    )

  return agg_kernel


_deg_call = _make_deg()
_agg_call = _make_agg()


R_BLK = 2000
GRID = N // R_BLK


def _tc_mm_body(x_ref, w_ref, xw_ref):
  xw_ref[...] = jnp.dot(
      x_ref[...], w_ref[...], preferred_element_type=jnp.float32
  )


def _tc_mm(x, w1):
  return pl.pallas_call(
      _tc_mm_body,
      grid=(GRID,),
      in_specs=[
          pl.BlockSpec((R_BLK, D), lambda i: (i, 0)),
          pl.BlockSpec((D, D), lambda i: (0, 0)),
      ],
      out_specs=pl.BlockSpec((R_BLK, D), lambda i: (i, 0)),
      out_shape=jax.ShapeDtypeStruct((N, D), jnp.float32),
  )(x, w1)


def _tc_scale_body(xw_ref, dis_ref, y_ref):
  y_ref[...] = xw_ref[...] * dis_ref[...]


def _tc_scale(xw, dis2d):
  return pl.pallas_call(
      _tc_scale_body,
      grid=(GRID,),
      in_specs=[
          pl.BlockSpec((R_BLK, D), lambda i: (i, 0)),
          pl.BlockSpec((R_BLK, 1), lambda i: (i, 0)),
      ],
      out_specs=pl.BlockSpec((R_BLK, D), lambda i: (i, 0)),
      out_shape=jax.ShapeDtypeStruct((N, D), jnp.float32),
  )(xw, dis2d)


def _tc2_body(p_ref, xw_ref, dis_ref, b_ref, w_ref, xw2_ref, y2_ref):
  dis = dis_ref[...]
  agg = p_ref[0] + p_ref[1]
  h = jnp.maximum(dis * agg + (dis * dis) * xw_ref[...] + b_ref[...], 0.0)
  xw2 = jnp.dot(h, w_ref[...], preferred_element_type=jnp.float32)
  xw2_ref[...] = xw2
  y2_ref[...] = xw2 * dis


def _tc2(p, xw1, dis2d, b1, w2):
  return pl.pallas_call(
      _tc2_body,
      grid=(GRID,),
      in_specs=[
          pl.BlockSpec((NC, R_BLK, D), lambda i: (0, i, 0)),
          pl.BlockSpec((R_BLK, D), lambda i: (i, 0)),
          pl.BlockSpec((R_BLK, 1), lambda i: (i, 0)),
          pl.BlockSpec((1, D), lambda i: (0, 0)),
          pl.BlockSpec((D, D), lambda i: (0, 0)),
      ],
      out_specs=[
          pl.BlockSpec((R_BLK, D), lambda i: (i, 0)),
          pl.BlockSpec((R_BLK, D), lambda i: (i, 0)),
      ],
      out_shape=[
          jax.ShapeDtypeStruct((N, D), jnp.float32),
          jax.ShapeDtypeStruct((N, D), jnp.float32),
      ],
  )(p, xw1, dis2d, b1, w2)


def _tc3_body(p_ref, xw_ref, dis_ref, b_ref, wout_ref, bout_ref, out_ref):
  dis = dis_ref[...]
  agg = p_ref[0] + p_ref[1]
  h = jnp.maximum(dis * agg + (dis * dis) * xw_ref[...] + b_ref[...], 0.0)
  logit = jnp.dot(h, wout_ref[...], preferred_element_type=jnp.float32)
  logit = logit + bout_ref[...]
  out_ref[...] = 1.0 / (1.0 + jnp.exp(-logit))


def _tc3(p, xw2, dis2d, b2, w_out, b_out):
  return pl.pallas_call(
      _tc3_body,
      grid=(GRID,),
      in_specs=[
          pl.BlockSpec((NC, R_BLK, D), lambda i: (0, i, 0)),
          pl.BlockSpec((R_BLK, D), lambda i: (i, 0)),
          pl.BlockSpec((R_BLK, 1), lambda i: (i, 0)),
          pl.BlockSpec((1, D), lambda i: (0, 0)),
          pl.BlockSpec((D, 1), lambda i: (0, 0)),
          pl.BlockSpec((1, 1), lambda i: (0, 0)),
      ],
      out_specs=pl.BlockSpec((R_BLK, 1), lambda i: (i, 0)),
      out_shape=jax.ShapeDtypeStruct((N, 1), jnp.float32),
  )(p, xw2, dis2d, b2, w_out, b_out)


def kernel(x, edge_index, W1, b1, W2, b2, W_out, b_out):
  src = edge_index[0].astype(jnp.int32)
  dst = edge_index[1].astype(jnp.int32)

  deg_p = _deg_call(dst)
  xw1 = _tc_mm(x, W1)
  deg = 1.0 + deg_p[0, :N, 0] + deg_p[1, :N, 0]
  dis2d = lax.rsqrt(deg)[:, None]

  y1 = _tc_scale(xw1, dis2d)
  p1 = _agg_call(y1, src, dst)
  xw2, y2 = _tc2(p1, xw1, dis2d, b1.reshape(1, D), W2)
  p2 = _agg_call(y2, src, dst)
  out = _tc3(p2, xw2, dis2d, b2.reshape(1, D), W_out, b_out.reshape(1, 1))
  return out[:, 0]


# speedup vs baseline: 30.3782x; 1.0145x over previous
import functools

import jax
import jax.numpy as jnp
from jax import lax
from jax.experimental import pallas as pl
from jax.experimental.pallas import tpu as pltpu
from jax.experimental.pallas import tpu_sc as plsc

N = 10000
E = 320000
D = 128

NC = 2
NS = 16
NW = NC * NS

E_PER_TILE = E // NW
CHUNK = 80
STEPS = E_PER_TILE // CHUNK

N_ROWS = 10240
ROWS_PER_TILE = N_ROWS // NS
ZCHUNK = 32

_MESH = plsc.VectorSubcoreMesh(core_axis_name="c", subcore_axis_name="s")


def _zero_vmem_2d(ref, rows, cols):
  zero = jnp.zeros((16,), jnp.float32)

  def body(r, carry):
    for j in range(cols // 16):
      ref[r, pl.ds(j * 16, 16)] = zero
    return carry

  lax.fori_loop(0, rows, body, 0)


def _zero_acc(zrow_v, zch, acc_sh, row_base, nrows, zsem):
  ncopies = nrows // zch
  k = 0
  while k < ncopies:
    batchn = min(5, ncopies - k)
    for t in range(batchn):
      pltpu.async_copy(
          zrow_v, acc_sh.at[pl.ds(row_base + (k + t) * zch, zch)], zsem
      )
    for t in range(batchn):
      pltpu.make_async_copy(
          zrow_v, acc_sh.at[pl.ds(row_base, zch)], zsem
      ).wait()
    k += batchn


DEG_CHUNK = 80
DEG_STEPS = E_PER_TILE // DEG_CHUNK
DEG_NBUF = 5


def _make_deg():
  scratch = (
      [pltpu.VMEM((DEG_CHUNK,), jnp.int32) for _ in range(DEG_NBUF)]
      + [
          pltpu.VMEM((DEG_CHUNK, D), jnp.float32),
          pltpu.VMEM((128, D), jnp.float32),
          pltpu.VMEM_SHARED((N_ROWS, D), jnp.float32),
      ]
      + [pltpu.SemaphoreType.DMA for _ in range(2 * DEG_NBUF + 1)]
  )

  @functools.partial(
      pl.kernel,
      out_type=jax.ShapeDtypeStruct((NC, N_ROWS, D), jnp.float32),
      mesh=_MESH,
      scratch_types=scratch,
  )
  def deg_kernel(dst_hbm, out_hbm, *refs):
    dstv = refs[0:DEG_NBUF]
    ones_v = refs[DEG_NBUF]
    zrow_v = refs[DEG_NBUF + 1]
    acc_sh = refs[DEG_NBUF + 2]
    isem = refs[DEG_NBUF + 3:DEG_NBUF + 3 + DEG_NBUF]
    ssem = refs[DEG_NBUF + 3 + DEG_NBUF:DEG_NBUF + 3 + 2 * DEG_NBUF]
    zsem = refs[DEG_NBUF + 3 + 2 * DEG_NBUF]

    c = lax.axis_index("c")
    s = lax.axis_index("s")
    wid = s * NC + c

    one = jnp.full((16,), 1.0, jnp.float32)

    def fill(r, carry):
      for j in range(D // 16):
        ones_v[r, pl.ds(j * 16, 16)] = one
      return carry

    lax.fori_loop(0, DEG_CHUNK, fill, 0)
    _zero_vmem_2d(zrow_v, 128, D)
    _zero_acc(zrow_v, 128, acc_sh, s * ROWS_PER_TILE, ROWS_PER_TILE, zsem)
    plsc.subcore_barrier()

    base = wid * E_PER_TILE

    def i_start(i, b):
      pltpu.async_copy(
          dst_hbm.at[pl.ds(base + i * DEG_CHUNK, DEG_CHUNK)], dstv[b], isem[b]
      )

    def iw_ss(i, b):
      pltpu.make_async_copy(
          dst_hbm.at[pl.ds(base + i * DEG_CHUNK, DEG_CHUNK)], dstv[b], isem[b]
      ).wait()
      pltpu.async_copy(ones_v, acc_sh.at[dstv[b]], ssem[b], add=True)

    def s_wait(b):
      pltpu.make_async_copy(ones_v, acc_sh.at[dstv[b]], ssem[b]).wait()

    for t in range(DEG_NBUF):
      i_start(t, t)
      if t >= 1:
        iw_ss(t - 1, t - 1)

    def batch(j, carry):
      for b in range(DEG_NBUF):
        t = j * DEG_NBUF + b
        s_wait(b)
        i_start(t, b)
        iw_ss(t - 1, (b + DEG_NBUF - 1) % DEG_NBUF)
      return carry

    lax.fori_loop(1, DEG_STEPS // DEG_NBUF, batch, 0)

    for t in range(DEG_STEPS, DEG_STEPS + DEG_NBUF):
      s_wait((t - DEG_NBUF) % DEG_NBUF)
      if t - 1 < DEG_STEPS:
        iw_ss(t - 1, (t - 1) % DEG_NBUF)

    plsc.subcore_barrier()

    pltpu.sync_copy(
        acc_sh.at[pl.ds(s * ROWS_PER_TILE, ROWS_PER_TILE)],
        out_hbm.at[c, pl.ds(s * ROWS_PER_TILE, ROWS_PER_TILE)],
    )

  return deg_kernel


NBUF = 4


def _make_agg():
  scratch = (
      [pltpu.VMEM((CHUNK,), jnp.int32) for _ in range(NBUF)]
      + [pltpu.VMEM((CHUNK,), jnp.int32) for _ in range(NBUF)]
      + [pltpu.VMEM((CHUNK, D), jnp.float32) for _ in range(NBUF)]
      + [pltpu.VMEM((ZCHUNK, D), jnp.float32)]
      + [pltpu.VMEM_SHARED((N_ROWS, D), jnp.float32)]
      + [pltpu.SemaphoreType.DMA for _ in range(3 * NBUF + 1)]
  )

  @functools.partial(
      pl.kernel,
      out_type=jax.ShapeDtypeStruct((NC, N_ROWS, D), jnp.float32),
      mesh=_MESH,
      scratch_types=scratch,
  )
  def agg_kernel(y_hbm, src_hbm, dst_hbm, out_hbm, *refs):
    srcv = refs[0:NBUF]
    dstv = refs[NBUF:2 * NBUF]
    rows = refs[2 * NBUF:3 * NBUF]
    zrow_v = refs[3 * NBUF]
    acc_sh = refs[3 * NBUF + 1]
    isem = refs[3 * NBUF + 2:3 * NBUF + 2 + NBUF]
    gsem = refs[3 * NBUF + 2 + NBUF:3 * NBUF + 2 + 2 * NBUF]
    ssem = refs[3 * NBUF + 2 + 2 * NBUF:3 * NBUF + 2 + 3 * NBUF]
    zsem = refs[3 * NBUF + 2 + 3 * NBUF]

    c = lax.axis_index("c")
    s = lax.axis_index("s")
    wid = s * NC + c

    _zero_vmem_2d(zrow_v, ZCHUNK, D)
    _zero_acc(zrow_v, ZCHUNK, acc_sh, s * ROWS_PER_TILE, ROWS_PER_TILE, zsem)
    plsc.subcore_barrier()

    base = wid * E_PER_TILE

    def i_start(i, b):
      off = base + i * CHUNK
      pltpu.async_copy(src_hbm.at[pl.ds(off, CHUNK)], srcv[b], isem[b])
      pltpu.async_copy(dst_hbm.at[pl.ds(off, CHUNK)], dstv[b], isem[b])

    def iw_gs(i, b):
      off = base + i * CHUNK
      pltpu.make_async_copy(src_hbm.at[pl.ds(off, CHUNK)], srcv[b], isem[b]).wait()
      pltpu.make_async_copy(dst_hbm.at[pl.ds(off, CHUNK)], dstv[b], isem[b]).wait()
      pltpu.async_copy(y_hbm.at[srcv[b]], rows[b], gsem[b])

    def gw_ss(i, b):
      pltpu.make_async_copy(y_hbm.at[srcv[b]], rows[b], gsem[b]).wait()
      pltpu.async_copy(rows[b], acc_sh.at[dstv[b]], ssem[b], add=True)

    def s_wait(i, b):
      pltpu.make_async_copy(rows[b], acc_sh.at[dstv[b]], ssem[b]).wait()

    def full_step(t, b):
      s_wait(t - NBUF, b)
      i_start(t, b)
      iw_gs(t - 1, (b + NBUF - 1) % NBUF)
      gw_ss(t - 2, (b + NBUF - 2) % NBUF)

    for t in range(NBUF):
      i_start(t, t % NBUF)
      if t - 1 >= 0:
        iw_gs(t - 1, (t - 1) % NBUF)
      if t - 2 >= 0:
        gw_ss(t - 2, (t - 2) % NBUF)

    def batch(j, carry):
      for b in range(NBUF):
        full_step(j * NBUF + b, b)
      return carry

    lax.fori_loop(1, STEPS // NBUF, batch, 0)

    for t in range((STEPS // NBUF) * NBUF, STEPS):
      full_step(t, t % NBUF)
    for t in range(STEPS, STEPS + NBUF):
      s_wait(t - NBUF, (t - NBUF) % NBUF)
      if t - 1 < STEPS:
        iw_gs(t - 1, (t - 1) % NBUF)
      if t - 2 < STEPS:
        gw_ss(t - 2, (t - 2) % NBUF)

    plsc.subcore_barrier()

    pltpu.sync_copy(
        acc_sh.at[pl.ds(s * ROWS_PER_TILE, ROWS_PER_TILE)],
        out_hbm.at[c, pl.ds(s * ROWS_PER_TILE, ROWS_PER_TILE)],
    )

  return agg_kernel


_deg_call = _make_deg()
_agg_call = _make_agg()


R_BLK = 5000
GRID = N // R_BLK


def _tc_mm_body(x_ref, w_ref, xw_ref):
  xw_ref[...] = jnp.dot(
      x_ref[...], w_ref[...], preferred_element_type=jnp.float32
  )


def _tc_mm(x, w1):
  return pl.pallas_call(
      _tc_mm_body,
      grid=(GRID,),
      in_specs=[
          pl.BlockSpec((R_BLK, D), lambda i: (i, 0)),
          pl.BlockSpec((D, D), lambda i: (0, 0)),
      ],
      out_specs=pl.BlockSpec((R_BLK, D), lambda i: (i, 0)),
      out_shape=jax.ShapeDtypeStruct((N, D), jnp.float32),
  )(x, w1)


def _tc_scale_body(xw_ref, dis_ref, y_ref):
  y_ref[...] = xw_ref[...] * dis_ref[...]


def _tc_scale(xw, dis2d):
  return pl.pallas_call(
      _tc_scale_body,
      grid=(GRID,),
      in_specs=[
          pl.BlockSpec((R_BLK, D), lambda i: (i, 0)),
          pl.BlockSpec((R_BLK, 1), lambda i: (i, 0)),
      ],
      out_specs=pl.BlockSpec((R_BLK, D), lambda i: (i, 0)),
      out_shape=jax.ShapeDtypeStruct((N, D), jnp.float32),
  )(xw, dis2d)


def _tc2_body(p_ref, xw_ref, dis_ref, b_ref, w_ref, xw2_ref, y2_ref):
  dis = dis_ref[...]
  agg = p_ref[0] + p_ref[1]
  h = jnp.maximum(dis * agg + (dis * dis) * xw_ref[...] + b_ref[...], 0.0)
  xw2 = jnp.dot(h, w_ref[...], preferred_element_type=jnp.float32)
  xw2_ref[...] = xw2
  y2_ref[...] = xw2 * dis


def _tc2(p, xw1, dis2d, b1, w2):
  return pl.pallas_call(
      _tc2_body,
      grid=(GRID,),
      in_specs=[
          pl.BlockSpec((NC, R_BLK, D), lambda i: (0, i, 0)),
          pl.BlockSpec((R_BLK, D), lambda i: (i, 0)),
          pl.BlockSpec((R_BLK, 1), lambda i: (i, 0)),
          pl.BlockSpec((1, D), lambda i: (0, 0)),
          pl.BlockSpec((D, D), lambda i: (0, 0)),
      ],
      out_specs=[
          pl.BlockSpec((R_BLK, D), lambda i: (i, 0)),
          pl.BlockSpec((R_BLK, D), lambda i: (i, 0)),
      ],
      out_shape=[
          jax.ShapeDtypeStruct((N, D), jnp.float32),
          jax.ShapeDtypeStruct((N, D), jnp.float32),
      ],
  )(p, xw1, dis2d, b1, w2)


def _tc3_body(p_ref, xw_ref, dis_ref, b_ref, wout_ref, bout_ref, out_ref):
  dis = dis_ref[...]
  agg = p_ref[0] + p_ref[1]
  h = jnp.maximum(dis * agg + (dis * dis) * xw_ref[...] + b_ref[...], 0.0)
  logit = jnp.dot(h, wout_ref[...], preferred_element_type=jnp.float32)
  logit = logit + bout_ref[...]
  out_ref[...] = 1.0 / (1.0 + jnp.exp(-logit))


def _tc3(p, xw2, dis2d, b2, w_out, b_out):
  return pl.pallas_call(
      _tc3_body,
      grid=(GRID,),
      in_specs=[
          pl.BlockSpec((NC, R_BLK, D), lambda i: (0, i, 0)),
          pl.BlockSpec((R_BLK, D), lambda i: (i, 0)),
          pl.BlockSpec((R_BLK, 1), lambda i: (i, 0)),
          pl.BlockSpec((1, D), lambda i: (0, 0)),
          pl.BlockSpec((D, 1), lambda i: (0, 0)),
          pl.BlockSpec((1, 1), lambda i: (0, 0)),
      ],
      out_specs=pl.BlockSpec((R_BLK, 1), lambda i: (i, 0)),
      out_shape=jax.ShapeDtypeStruct((N, 1), jnp.float32),
  )(p, xw2, dis2d, b2, w_out, b_out)


def kernel(x, edge_index, W1, b1, W2, b2, W_out, b_out):
  src = edge_index[0].astype(jnp.int32)
  dst = edge_index[1].astype(jnp.int32)

  deg_p = _deg_call(dst)
  xw1 = _tc_mm(x, W1)
  deg = 1.0 + deg_p[0, :N, 0] + deg_p[1, :N, 0]
  dis2d = lax.rsqrt(deg)[:, None]

  y1 = _tc_scale(xw1, dis2d)
  p1 = _agg_call(y1, src, dst)
  xw2, y2 = _tc2(p1, xw1, dis2d, b1.reshape(1, D), W2)
  p2 = _agg_call(y2, src, dst)
  out = _tc3(p2, xw2, dis2d, b2.reshape(1, D), W_out, b_out.reshape(1, 1))
  return out[:, 0]
